# Initial kernel scaffold; baseline (speedup 1.0000x reference)
import jax
import jax.numpy as jnp
from jax import lax
from jax.experimental import pallas as pl
from jax.experimental.pallas import tpu as pltpu
from jax.experimental.pallas import tpu_sc as plsc

N = 100000
E = 6400000
NC = 2
NS = 16
L = 16
W = NC * NS
E_W = E // W
C = 2000
NCH = E_W // C
ACC_N = 100096
SLICE = ACC_N // NS


def _sc_edge_pass(x, edge_index, edge_weight):
    mesh = plsc.VectorSubcoreMesh(core_axis_name="c", subcore_axis_name="s")

    @pl.kernel(
        out_type=jax.ShapeDtypeStruct((NC, ACC_N), jnp.float32),
        mesh=mesh,
        scratch_types=[
            pltpu.VMEM((N,), jnp.float32),
            pltpu.VMEM((C,), jnp.int32),
            pltpu.VMEM((C,), jnp.int32),
            pltpu.VMEM((C,), jnp.float32),
            pltpu.VMEM((C,), jnp.float32),
            pltpu.VMEM((SLICE,), jnp.float32),
            pltpu.VMEM_SHARED((ACC_N,), jnp.float32),
        ],
    )
    def edge_pass(x_hbm, ei_hbm, w_hbm, out_hbm,
                  state_v, src_v, dst_v, w_v, msg_v, zero_v, acc_sh):
        cid = lax.axis_index("c")
        sid = lax.axis_index("s")
        wid = sid * NC + cid
        base = wid * E_W

        pltpu.sync_copy(x_hbm, state_v)

        def zero_body(i, _):
            zero_v[pl.ds(i * L, L)] = jnp.zeros((L,), jnp.float32)
            return 0
        lax.fori_loop(0, SLICE // L, zero_body, 0)
        pltpu.sync_copy(zero_v, acc_sh.at[pl.ds(sid * SLICE, SLICE)])
        plsc.subcore_barrier()

        def chunk_body(k, _):
            off = base + k * C
            pltpu.sync_copy(ei_hbm.at[0, pl.ds(off, C)], src_v)
            pltpu.sync_copy(ei_hbm.at[1, pl.ds(off, C)], dst_v)
            pltpu.sync_copy(w_hbm.at[pl.ds(off, C)], w_v)

            def gather_body(j, _):
                sl = pl.ds(j * L, L)
                vals = plsc.load_gather(state_v, [src_v[sl]])
                msg_v[sl] = w_v[sl] * vals
                return 0
            lax.fori_loop(0, C // L, gather_body, 0)

            pltpu.sync_copy(msg_v, acc_sh.at[dst_v], add=True)
            return 0
        lax.fori_loop(0, NCH, chunk_body, 0)

        plsc.subcore_barrier()
        sl = pl.ds(sid * SLICE, SLICE)
        pltpu.sync_copy(acc_sh.at[sl], out_hbm.at[cid, sl])

    return edge_pass(x, edge_index, edge_weight)


def _tc_finish(x2d, p02d, p12d):
    def finish(x_ref, p0_ref, p1_ref, o_ref):
        o_ref[...] = jnp.tanh(x_ref[...] + p0_ref[...] + p1_ref[...])

    return pl.pallas_call(
        finish,
        out_shape=jax.ShapeDtypeStruct(x2d.shape, jnp.float32),
    )(x2d, p02d, p12d)


def kernel(region_inputs_flat, edge_index, edge_weight):
    x = region_inputs_flat
    partials = _sc_edge_pass(x, edge_index.astype(jnp.int32), edge_weight)
    x2d = x.reshape(8, N // 8)
    p0 = partials[0, :N].reshape(8, N // 8)
    p1 = partials[1, :N].reshape(8, N // 8)
    return _tc_finish(x2d, p0, p1).reshape(N)


# speedup vs baseline: 169.1881x; 169.1881x over previous
import jax
import jax.numpy as jnp
from jax import lax
from jax.experimental import pallas as pl
from jax.experimental.pallas import tpu as pltpu
from jax.experimental.pallas import tpu_sc as plsc

N = 100000
E = 6400000
NC = 2
NS = 16
L = 16
W = NC * NS
E_W = E // W
C = 2000
NCH = E_W // C
ACC_N = 100096
SLICE = ACC_N // NS


def _sc_edge_pass(x, edge_index, edge_weight):
    mesh = plsc.VectorSubcoreMesh(core_axis_name="c", subcore_axis_name="s")

    @pl.kernel(
        out_type=jax.ShapeDtypeStruct((NC * ACC_N,), jnp.float32),
        mesh=mesh,
        compiler_params=pltpu.CompilerParams(needs_layout_passes=False),
        scratch_types=[
            pltpu.VMEM((N,), jnp.float32),
            pltpu.VMEM((C,), jnp.int32),
            pltpu.VMEM((C,), jnp.int32),
            pltpu.VMEM((C,), jnp.float32),
            pltpu.VMEM((C,), jnp.float32),
            pltpu.VMEM((SLICE,), jnp.float32),
            pltpu.VMEM_SHARED((ACC_N,), jnp.float32),
        ],
    )
    def edge_pass(x_hbm, ei_hbm, w_hbm, out_hbm,
                  state_v, src_v, dst_v, w_v, msg_v, zero_v, acc_sh):
        cid = lax.axis_index("c")
        sid = lax.axis_index("s")
        wid = sid * NC + cid
        base = wid * E_W

        pltpu.sync_copy(x_hbm, state_v)

        def zero_body(i, _):
            zero_v[pl.ds(i * L, L)] = jnp.zeros((L,), jnp.float32)
            return 0
        lax.fori_loop(0, SLICE // L, zero_body, 0)
        pltpu.sync_copy(zero_v, acc_sh.at[pl.ds(sid * SLICE, SLICE)])
        plsc.subcore_barrier()

        def chunk_body(k, _):
            off = base + k * C
            pltpu.sync_copy(ei_hbm.at[pl.ds(off, C)], src_v)
            pltpu.sync_copy(ei_hbm.at[pl.ds(E + off, C)], dst_v)
            pltpu.sync_copy(w_hbm.at[pl.ds(off, C)], w_v)

            def gather_body(j, _):
                sl = pl.ds(j * L, L)
                vals = plsc.load_gather(state_v, [src_v[sl]])
                msg_v[sl] = w_v[sl] * vals
                return 0
            lax.fori_loop(0, C // L, gather_body, 0)

            pltpu.sync_copy(msg_v, acc_sh.at[dst_v], add=True)
            return 0
        lax.fori_loop(0, NCH, chunk_body, 0)

        plsc.subcore_barrier()
        pltpu.sync_copy(acc_sh.at[pl.ds(sid * SLICE, SLICE)], zero_v)
        pltpu.sync_copy(zero_v,
                        out_hbm.at[pl.ds(cid * ACC_N + sid * SLICE, SLICE)])

    return edge_pass(x, edge_index, edge_weight)


def _tc_finish(x2d, p02d, p12d):
    def finish(x_ref, p0_ref, p1_ref, o_ref):
        o_ref[...] = jnp.tanh(x_ref[...] + p0_ref[...] + p1_ref[...])

    return pl.pallas_call(
        finish,
        out_shape=jax.ShapeDtypeStruct(x2d.shape, jnp.float32),
    )(x2d, p02d, p12d)


def kernel(region_inputs_flat, edge_index, edge_weight):
    x = region_inputs_flat
    ei_flat = edge_index.astype(jnp.int32).reshape(2 * E)
    partials = _sc_edge_pass(x, ei_flat, edge_weight)
    x2d = x.reshape(8, N // 8)
    p0 = partials[:N].reshape(8, N // 8)
    p1 = partials[ACC_N:ACC_N + N].reshape(8, N // 8)
    return _tc_finish(x2d, p0, p1).reshape(N)


# speedup vs baseline: 321.6628x; 1.9012x over previous
import jax
import jax.numpy as jnp
from jax import lax
from jax.experimental import pallas as pl
from jax.experimental.pallas import tpu as pltpu
from jax.experimental.pallas import tpu_sc as plsc

N = 100000
E = 6400000
NC = 2
NS = 16
L = 16
W = NC * NS
E_W = E // W
C = 2000
NCH = E_W // C
D = 3
ACC_N = 102400
SLICE = ACC_N // NS
UNROLL = 5
_PIECES = [(0, 2000), (2000, 2000), (4000, 2000), (6000, 400)]


def _sc_edge_pass(x, edge_index, edge_weight):
    mesh = plsc.VectorSubcoreMesh(core_axis_name="c", subcore_axis_name="s")

    @pl.kernel(
        out_type=jax.ShapeDtypeStruct((NC * ACC_N,), jnp.float32),
        mesh=mesh,
        compiler_params=pltpu.CompilerParams(needs_layout_passes=False),
        scratch_types=[
            pltpu.VMEM((N,), jnp.float32),
            [pltpu.VMEM((C,), jnp.int32) for _ in range(D)],
            [pltpu.VMEM((C,), jnp.int32) for _ in range(D)],
            [pltpu.VMEM((C,), jnp.float32) for _ in range(D)],
            [pltpu.VMEM((C,), jnp.float32) for _ in range(D)],
            [pltpu.SemaphoreType.DMA for _ in range(D)],
            [pltpu.SemaphoreType.DMA for _ in range(D)],
            pltpu.VMEM_SHARED((ACC_N,), jnp.float32),
        ],
    )
    def edge_pass(x_hbm, ei_hbm, w_hbm, out_hbm,
                  state_v, src_v, dst_v, w_v, msg_v, sem_in, sem_sc, acc_sh):
        cid = lax.axis_index("c")
        sid = lax.axis_index("s")
        wid = sid * NC + cid
        base = wid * E_W

        pltpu.sync_copy(x_hbm, state_v)

        def zero_body(i, _):
            msg_v[0][pl.ds(i * L, L)] = jnp.zeros((L,), jnp.float32)
            return 0
        lax.fori_loop(0, C // L, zero_body, 0)
        for o, ln in _PIECES:
            pltpu.sync_copy(msg_v[0].at[pl.ds(0, ln)],
                            acc_sh.at[pl.ds(sid * SLICE + o, ln)])
        plsc.subcore_barrier()

        def issue_in(b, off):
            return (
                pltpu.async_copy(ei_hbm.at[pl.ds(off, C)], src_v[b], sem_in[b]),
                pltpu.async_copy(ei_hbm.at[pl.ds(E + off, C)], dst_v[b], sem_in[b]),
                pltpu.async_copy(w_hbm.at[pl.ds(off, C)], w_v[b], sem_in[b]),
            )

        def compute(b):
            def gather_body(j, _):
                for u in range(UNROLL):
                    sl = pl.ds((j * UNROLL + u) * L, L)
                    vals = plsc.load_gather(state_v, [src_v[b][sl]])
                    msg_v[b][sl] = w_v[b][sl] * vals
                return 0
            lax.fori_loop(0, C // (L * UNROLL), gather_body, 0)

        def do_chunks(first):
            ins = [issue_in(b, base + (first + b) * C) for b in range(D)]
            scs = []
            for b in range(D):
                for d in ins[b]:
                    d.wait()
                compute(b)
                scs.append(pltpu.async_copy(
                    msg_v[b], acc_sh.at[dst_v[b]], sem_sc[b], add=True))
            for s in scs:
                s.wait()

        def group_body(kk, _):
            do_chunks(kk * D)
            return 0
        lax.fori_loop(0, NCH // D, group_body, 0)
        if NCH % D:
            ins = [issue_in(b, base + (NCH - NCH % D + b) * C)
                   for b in range(NCH % D)]
            scs = []
            for b in range(NCH % D):
                for d in ins[b]:
                    d.wait()
                compute(b)
                scs.append(pltpu.async_copy(
                    msg_v[b], acc_sh.at[dst_v[b]], sem_sc[b], add=True))
            for s in scs:
                s.wait()

        plsc.subcore_barrier()
        for o, ln in _PIECES:
            pltpu.sync_copy(acc_sh.at[pl.ds(sid * SLICE + o, ln)],
                            msg_v[0].at[pl.ds(0, ln)])
            pltpu.sync_copy(msg_v[0].at[pl.ds(0, ln)],
                            out_hbm.at[pl.ds(cid * ACC_N + sid * SLICE + o, ln)])

    return edge_pass(x, edge_index, edge_weight)


def _tc_finish(x2d, p02d, p12d):
    def finish(x_ref, p0_ref, p1_ref, o_ref):
        o_ref[...] = jnp.tanh(x_ref[...] + p0_ref[...] + p1_ref[...])

    return pl.pallas_call(
        finish,
        out_shape=jax.ShapeDtypeStruct(x2d.shape, jnp.float32),
    )(x2d, p02d, p12d)


def kernel(region_inputs_flat, edge_index, edge_weight):
    x = region_inputs_flat
    ei_flat = edge_index.astype(jnp.int32).reshape(2 * E)
    partials = _sc_edge_pass(x, ei_flat, edge_weight)
    x2d = x.reshape(8, N // 8)
    p0 = partials[:N].reshape(8, N // 8)
    p1 = partials[ACC_N:ACC_N + N].reshape(8, N // 8)
    return _tc_finish(x2d, p0, p1).reshape(N)
